# Initial kernel scaffold; baseline (speedup 1.0000x reference)
import jax
import jax.numpy as jnp
from jax.experimental import pallas as pl


def kernel(embeddings, cluster_labels, centers):
    raise NotImplementedError("write your pallas kernel here")


# speedup vs baseline: 1.5905x; 1.5905x over previous
import functools

import jax
import jax.numpy as jnp
from jax import lax
from jax.experimental import pallas as pl
from jax.experimental.pallas import tpu as pltpu
from jax.experimental.pallas import tpu_sc as plsc

_B = 16384
_D = 128
_K = 100
_ALPHA = 0.5
_EPS = 1e-6

_INFO = plsc.get_sparse_core_info()
_NC = _INFO.num_cores
_NS = _INFO.num_subcores
_NW = _NC * _NS
_RW = _B // _NW
_CH = 128
_NCHUNK = _RW // _CH

_mesh = plsc.VectorSubcoreMesh(core_axis_name="c", subcore_axis_name="s")


@functools.partial(
    pl.kernel,
    mesh=_mesh,
    out_type=jax.ShapeDtypeStruct((_NW, 16), jnp.float32),
    scratch_types=[
        pltpu.VMEM((_CH,), jnp.int32),
        pltpu.VMEM((_CH, _D), jnp.float32),
        pltpu.VMEM((_CH, _D), jnp.float32),
        pltpu.VMEM((16,), jnp.float32),
        pltpu.SemaphoreType.DMA,
    ],
)
def _center_partials(emb_hbm, lab_hbm, cen_hbm, out_hbm,
                     lab_v, emb_v, rows_v, acc_v, sem):
    wid = lax.axis_index("s") * _NC + lax.axis_index("c")
    base = wid * _RW
    acc = jnp.zeros((16,), jnp.float32)
    for chunk in range(_NCHUNK):
        r0 = base + chunk * _CH
        pltpu.sync_copy(lab_hbm.at[pl.ds(r0, _CH)], lab_v)
        pltpu.sync_copy(emb_hbm.at[pl.ds(r0, _CH)], emb_v)
        pltpu.async_copy(cen_hbm.at[lab_v], rows_v, sem).wait()

        def body(r, a):
            for g in range(_D // 16):
                e = emb_v[r, pl.ds(g * 16, 16)]
                c = rows_v[r, pl.ds(g * 16, 16)]
                d = e - c
                a = a + d * d
            return a

        acc = lax.fori_loop(0, _CH, body, acc)
    acc_v[...] = acc
    pltpu.sync_copy(acc_v, out_hbm.at[wid])


def _sep_and_combine(cen_ref, part_ref, out_ref):
    c = cen_ref[...]
    g = lax.dot_general(c, c, (((1,), (1,)), ((), ())),
                        precision=lax.Precision.HIGHEST)
    row = lax.broadcasted_iota(jnp.int32, (_K, _K), 0)
    col = lax.broadcasted_iota(jnp.int32, (_K, _K), 1)
    eye = jnp.where(row == col, 1.0, 0.0)
    n_col = jnp.sum(g * eye, axis=1, keepdims=True)
    n_row = jnp.sum(g * eye, axis=0, keepdims=True)
    s_col = jnp.sum(c, axis=1, keepdims=True)
    s_row = jnp.sum(eye * s_col, axis=0, keepdims=True)
    d2 = (n_col + n_row - 2.0 * g
          + (2.0 * _EPS) * (s_col - s_row) + _D * _EPS * _EPS)
    dist = jnp.sqrt(jnp.maximum(d2, 0.0)) * (1.0 - eye)
    sep_sum = jnp.sum(dist)
    center_sum = jnp.sum(part_ref[...])
    total = center_sum / (_B * _D) - _ALPHA * sep_sum / (_K * (_K - 1))
    out_ref[...] = jnp.reshape(total, (1, 1))


def kernel(embeddings, cluster_labels, centers):
    partials = _center_partials(embeddings, cluster_labels, centers)
    total = pl.pallas_call(
        _sep_and_combine,
        out_shape=jax.ShapeDtypeStruct((1, 1), jnp.float32),
    )(centers, partials.reshape(4, 128))
    return total.reshape(())


# speedup vs baseline: 2.1245x; 1.3357x over previous
import functools

import jax
import jax.numpy as jnp
from jax import lax
from jax.experimental import pallas as pl
from jax.experimental.pallas import tpu as pltpu
from jax.experimental.pallas import tpu_sc as plsc

_B = 16384
_D = 128
_K = 100
_ALPHA = 0.5
_EPS = 1e-6

_INFO = plsc.get_sparse_core_info()
_NC = _INFO.num_cores
_NS = _INFO.num_subcores
_NW = _NC * _NS
_RW = _B // _NW
_CH = 128
_NCHUNK = _RW // _CH

_mesh = plsc.VectorSubcoreMesh(core_axis_name="c", subcore_axis_name="s")


@functools.partial(
    pl.kernel,
    mesh=_mesh,
    out_type=jax.ShapeDtypeStruct((_NW, 16), jnp.float32),
    compiler_params=pltpu.CompilerParams(needs_layout_passes=False),
    scratch_types=[
        pltpu.VMEM((_RW,), jnp.int32),
        pltpu.VMEM((_K, _D), jnp.float32),
        pltpu.VMEM((_CH, _D), jnp.float32),
        pltpu.VMEM((_CH, _D), jnp.float32),
        pltpu.VMEM((16,), jnp.float32),
        pltpu.SemaphoreType.DMA,
        pltpu.SemaphoreType.DMA,
    ],
)
def _center_partials(emb_hbm, lab_hbm, cen_hbm, out_hbm,
                     lab_v, tab_v, emb0, emb1, acc_v, sem0, sem1):
    wid = lax.axis_index("s") * _NC + lax.axis_index("c")
    base = wid * _RW
    pltpu.sync_copy(cen_hbm, tab_v)
    pltpu.sync_copy(lab_hbm.at[pl.ds(base, _RW)], lab_v)

    bufs = (emb0, emb1)
    sems = (sem0, sem1)
    copies = [None, None]
    copies[0] = pltpu.async_copy(emb_hbm.at[pl.ds(base, _CH)], emb0, sem0)

    cols = [lax.iota(jnp.int32, 16) + g * 16 for g in range(_D // 16)]
    acc = jnp.zeros((16,), jnp.float32)
    for chunk in range(_NCHUNK):
        cur = chunk % 2
        nxt = 1 - cur
        if chunk + 1 < _NCHUNK:
            copies[nxt] = pltpu.async_copy(
                emb_hbm.at[pl.ds(base + (chunk + 1) * _CH, _CH)],
                bufs[nxt], sems[nxt])
        copies[cur].wait()
        ebuf = bufs[cur]
        row0 = chunk * _CH

        def body(r, a, ebuf=ebuf, row0=row0):
            lbl = plsc.load_gather(lab_v, [jnp.full((16,), row0 + r, jnp.int32)])
            for g in range(_D // 16):
                c = plsc.load_gather(tab_v, [lbl, cols[g]])
                e = ebuf[r, pl.ds(g * 16, 16)]
                d = e - c
                a = a + d * d
            return a

        acc = lax.fori_loop(0, _CH, body, acc)
    acc_v[...] = acc
    pltpu.sync_copy(acc_v, out_hbm.at[wid])


def _sep_kernel(cen_ref, out_ref):
    c = cen_ref[...]
    g = lax.dot_general(c, c, (((1,), (1,)), ((), ())),
                        precision=lax.Precision.HIGHEST)
    row = lax.broadcasted_iota(jnp.int32, (_K, _K), 0)
    col = lax.broadcasted_iota(jnp.int32, (_K, _K), 1)
    eye = jnp.where(row == col, 1.0, 0.0)
    n_col = jnp.sum(g * eye, axis=1, keepdims=True)
    n_row = jnp.sum(g * eye, axis=0, keepdims=True)
    s_col = jnp.sum(c, axis=1, keepdims=True)
    s_row = jnp.sum(eye * s_col, axis=0, keepdims=True)
    d2 = (n_col + n_row - 2.0 * g
          + (2.0 * _EPS) * (s_col - s_row) + _D * _EPS * _EPS)
    dist = jnp.sqrt(jnp.maximum(d2, 0.0)) * (1.0 - eye)
    out_ref[...] = jnp.reshape(jnp.sum(dist), (1, 1))


def kernel(embeddings, cluster_labels, centers):
    partials = _center_partials(embeddings, cluster_labels, centers)
    sep = pl.pallas_call(
        _sep_kernel,
        out_shape=jax.ShapeDtypeStruct((1, 1), jnp.float32),
    )(centers)
    total = jnp.sum(partials) / (_B * _D) - _ALPHA * sep[0, 0] / (_K * (_K - 1))
    return total
